# Initial kernel scaffold; baseline (speedup 1.0000x reference)
import jax
import jax.numpy as jnp
from jax.experimental import pallas as pl


def kernel(x, rotation_angles, mean_correction, centroids, running_mean):
    raise NotImplementedError("write your pallas kernel here")


# speedup vs baseline: 2.5643x; 2.5643x over previous
import functools

import jax
import jax.numpy as jnp
from jax import lax
from jax.experimental import pallas as pl
from jax.experimental.pallas import tpu as pltpu
from jax.experimental.pallas import tpu_sc as plsc

_NC = 2
_NS = 16
_NW = _NC * _NS
_L = 16
_D = 128
_KPR = _D // _L
_NLEV = 8

_OFF_C = 0
_OFF_G = 128
_OFF_MEAN = 256
_OFF_B = 384
_OFF_C0 = 496
_OFF_DELTA = 512
_PREP_LEN = 640


def _lane_perm(v, xor_mask):
    perm = lax.iota(jnp.int32, _L) ^ xor_mask
    return lax.gather(
        v, perm[:, None],
        lax.GatherDimensionNumbers(
            offset_dims=(), collapsed_slice_dims=(0,), start_index_map=(0,)),
        (1,), mode=lax.GatherScatterMode.PROMISE_IN_BOUNDS)


def _swap_pairs(v):
    return _lane_perm(v, 1)


def _lane_sum_splat(v):
    for step in (1, 2, 4, 8):
        v = v + _lane_perm(v, step)
    return v


def _rsqrt_newton(x):
    i = plsc.bitcast(x, jnp.int32)
    i = 0x5F3759DF - (i >> 1)
    y = plsc.bitcast(i, jnp.float32)
    xh = x * 0.5
    for _ in range(3):
        y = y * (1.5 - xh * y * y)
    return y


def _build_prep(rotation_angles, mean_correction, centroids, running_mean):
    c = jnp.cos(rotation_angles)
    s = jnp.sin(rotation_angles)
    cfull = jnp.repeat(c, 2)
    gfull = jnp.stack([-s, s], axis=-1).reshape(-1)
    mean = running_mean + mean_correction
    bounds = (centroids[:-1] + centroids[1:]) * 0.5
    deltas = centroids[1:] - centroids[:-1]
    row3 = jnp.concatenate([jnp.repeat(bounds, _L), jnp.full((_L,), centroids[0])])
    row4 = jnp.concatenate([jnp.repeat(deltas, _L), jnp.zeros((_L,))])
    return jnp.concatenate([cfull, gfull, mean, row3, row4]).astype(jnp.float32)


def _make_sc_kernel(n_tokens, block_rows):
    rows_per_w = n_tokens // _NW
    n_blocks = rows_per_w // block_rows

    @functools.partial(
        pl.kernel,
        out_type=jax.ShapeDtypeStruct((n_tokens, _D), jnp.float32),
        mesh=plsc.VectorSubcoreMesh(core_axis_name="c", subcore_axis_name="s"),
        compiler_params=pltpu.CompilerParams(needs_layout_passes=False),
        scratch_types=[
            pltpu.VMEM((block_rows, _D), jnp.float32),
            pltpu.VMEM((block_rows, _D), jnp.float32),
            pltpu.VMEM((_PREP_LEN,), jnp.float32),
        ],
    )
    def sc_kernel(x_hbm, prep_hbm, out_hbm, in_v, out_v, prep_v):
        wid = lax.axis_index("s") * _NC + lax.axis_index("c")
        base = wid * rows_per_w
        pltpu.sync_copy(prep_hbm, prep_v)

        def row_body(r, carry):
            xs = []
            acc = None
            for k in range(_KPR):
                mk = prep_v[pl.ds(_OFF_MEAN + _L * k, _L)]
                v = in_v[r, pl.ds(_L * k, _L)] - mk
                xs.append(v)
                acc = v * v if acc is None else acc + v * v
            totv = jnp.maximum(_lane_sum_splat(acc), 1e-16)
            inv = _rsqrt_newton(totv)
            norm = totv * inv
            zero = jnp.zeros((_L,), jnp.float32)
            for k in range(_KPR):
                ck = prep_v[pl.ds(_OFF_C + _L * k, _L)]
                gk = prep_v[pl.ds(_OFF_G + _L * k, _L)]
                mk = prep_v[pl.ds(_OFF_MEAN + _L * k, _L)]
                v = xs[k] * inv
                w = _swap_pairs(v)
                rot = ck * v + gk * w
                q = prep_v[pl.ds(_OFF_C0, _L)]
                for i in range(_NLEV - 1):
                    bi = prep_v[pl.ds(_OFF_B + _L * i, _L)]
                    di = prep_v[pl.ds(_OFF_DELTA + _L * i, _L)]
                    q = q + jnp.where(rot > bi, di, zero)
                sq = _swap_pairs(q)
                recon = ck * q - gk * sq
                out_v[r, pl.ds(_L * k, _L)] = recon * norm + mk
            return carry

        def blk_body(b, carry):
            r0 = base + b * block_rows
            pltpu.sync_copy(x_hbm.at[pl.ds(r0, block_rows)], in_v)
            lax.fori_loop(0, block_rows, row_body, 0, unroll=2)
            pltpu.sync_copy(out_v, out_hbm.at[pl.ds(r0, block_rows)])
            return carry

        lax.fori_loop(0, n_blocks, blk_body, 0)

    return sc_kernel


def kernel(x, rotation_angles, mean_correction, centroids, running_mean):
    x = x.astype(jnp.float32)
    n_tokens = x.shape[0]
    prep = _build_prep(rotation_angles, mean_correction, centroids, running_mean)
    return _make_sc_kernel(n_tokens, 128)(x, prep)


# speedup vs baseline: 2.6638x; 1.0388x over previous
import functools

import jax
import jax.numpy as jnp
from jax import lax
from jax.experimental import pallas as pl
from jax.experimental.pallas import tpu as pltpu
from jax.experimental.pallas import tpu_sc as plsc

_NC = 2
_NS = 16
_NW = _NC * _NS
_L = 16
_D = 128
_KPR = _D // _L
_NLEV = 8

_OFF_C = 0
_OFF_G = 128
_OFF_MEAN = 256
_OFF_BVEC = 384
_OFF_LVEC = 400
_OFF_B3 = 416
_PREP_LEN = 432


def _lane_gather(v, idx):
    return lax.gather(
        v, idx[:, None],
        lax.GatherDimensionNumbers(
            offset_dims=(), collapsed_slice_dims=(0,), start_index_map=(0,)),
        (1,), mode=lax.GatherScatterMode.PROMISE_IN_BOUNDS)


def _lane_perm(v, xor_mask):
    return _lane_gather(v, lax.iota(jnp.int32, _L) ^ xor_mask)


def _swap_pairs(v):
    return _lane_perm(v, 1)


def _lane_sum_splat(v):
    for step in (1, 2, 4, 8):
        v = v + _lane_perm(v, step)
    return v


def _rsqrt_newton(x):
    i = plsc.bitcast(x, jnp.int32)
    i = 0x5F3759DF - (i >> 1)
    y = plsc.bitcast(i, jnp.float32)
    xh = x * 0.5
    for _ in range(2):
        y = y * (1.5 - xh * y * y)
    return y


def _build_prep(rotation_angles, mean_correction, centroids, running_mean):
    c = jnp.cos(rotation_angles)
    s = jnp.sin(rotation_angles)
    cfull = jnp.repeat(c, 2)
    gfull = jnp.stack([-s, s], axis=-1).reshape(-1)
    mean = running_mean + mean_correction
    bounds = (centroids[:-1] + centroids[1:]) * 0.5
    bvec = jnp.concatenate([bounds, jnp.zeros((_L - _NLEV + 1,))])
    lvec = jnp.concatenate([centroids, jnp.zeros((_L - _NLEV,))])
    b3 = jnp.full((_L,), bounds[_NLEV // 2 - 1])
    return jnp.concatenate([cfull, gfull, mean, bvec, lvec, b3]).astype(jnp.float32)


def _make_sc_kernel(n_tokens, block_rows):
    rows_per_w = n_tokens // _NW
    n_blocks = rows_per_w // block_rows

    @functools.partial(
        pl.kernel,
        out_type=jax.ShapeDtypeStruct((n_tokens, _D), jnp.float32),
        mesh=plsc.VectorSubcoreMesh(core_axis_name="c", subcore_axis_name="s"),
        compiler_params=pltpu.CompilerParams(needs_layout_passes=False),
        scratch_types=[
            pltpu.VMEM((block_rows, _D), jnp.float32),
            pltpu.VMEM((block_rows, _D), jnp.float32),
            pltpu.VMEM((_PREP_LEN,), jnp.float32),
        ],
    )
    def sc_kernel(x_hbm, prep_hbm, out_hbm, in_v, out_v, prep_v):
        wid = lax.axis_index("s") * _NC + lax.axis_index("c")
        base = wid * rows_per_w
        pltpu.sync_copy(prep_hbm, prep_v)

        def row_body(r, carry):
            xs = []
            acc = None
            for k in range(_KPR):
                mk = prep_v[pl.ds(_OFF_MEAN + _L * k, _L)]
                v = in_v[r, pl.ds(_L * k, _L)] - mk
                xs.append(v)
                acc = v * v if acc is None else acc + v * v
            totv = jnp.maximum(_lane_sum_splat(acc), 1e-16)
            inv = _rsqrt_newton(totv)
            norm = totv * inv
            bvec = prep_v[pl.ds(_OFF_BVEC, _L)]
            lvec = prep_v[pl.ds(_OFF_LVEC, _L)]
            b3 = prep_v[pl.ds(_OFF_B3, _L)]
            zero = jnp.zeros((_L,), jnp.int32)
            for k in range(_KPR):
                ck = prep_v[pl.ds(_OFF_C + _L * k, _L)]
                gk = prep_v[pl.ds(_OFF_G + _L * k, _L)]
                mk = prep_v[pl.ds(_OFF_MEAN + _L * k, _L)]
                v = xs[k] * inv
                w = _swap_pairs(v)
                rot = ck * v + gk * w
                i0 = jnp.where(rot > b3, jnp.full((_L,), 4, jnp.int32), zero)
                b1v = _lane_gather(bvec, i0 + 1)
                i1 = i0 + jnp.where(rot > b1v, jnp.full((_L,), 2, jnp.int32), zero)
                b2v = _lane_gather(bvec, i1)
                i2 = i1 + jnp.where(rot > b2v, jnp.full((_L,), 1, jnp.int32), zero)
                q = _lane_gather(lvec, i2)
                sq = _swap_pairs(q)
                recon = ck * q - gk * sq
                out_v[r, pl.ds(_L * k, _L)] = recon * norm + mk
            return carry

        def blk_body(b, carry):
            r0 = base + b * block_rows
            pltpu.sync_copy(x_hbm.at[pl.ds(r0, block_rows)], in_v)
            lax.fori_loop(0, block_rows, row_body, 0, unroll=4)
            pltpu.sync_copy(out_v, out_hbm.at[pl.ds(r0, block_rows)])
            return carry

        lax.fori_loop(0, n_blocks, blk_body, 0)

    return sc_kernel


def kernel(x, rotation_angles, mean_correction, centroids, running_mean):
    x = x.astype(jnp.float32)
    n_tokens = x.shape[0]
    prep = _build_prep(rotation_angles, mean_correction, centroids, running_mean)
    return _make_sc_kernel(n_tokens, 128)(x, prep)


# speedup vs baseline: 2.9501x; 1.1075x over previous
import functools

import jax
import jax.numpy as jnp
from jax import lax
from jax.experimental import pallas as pl
from jax.experimental.pallas import tpu as pltpu
from jax.experimental.pallas import tpu_sc as plsc

_NC = 2
_NS = 16
_NW = _NC * _NS
_L = 16
_D = 128
_KPR = _D // _L
_NLEV = 8

_OFF_C = 0
_OFF_G = 128
_OFF_MEAN = 256
_OFF_BVEC = 384
_OFF_LVEC = 400
_OFF_B3 = 416
_PREP_LEN = 432


def _lane_gather(v, idx):
    return lax.gather(
        v, idx[:, None],
        lax.GatherDimensionNumbers(
            offset_dims=(), collapsed_slice_dims=(0,), start_index_map=(0,)),
        (1,), mode=lax.GatherScatterMode.PROMISE_IN_BOUNDS)


def _lane_perm(v, xor_mask):
    return _lane_gather(v, lax.iota(jnp.int32, _L) ^ xor_mask)


def _swap_pairs(v):
    return _lane_perm(v, 1)


def _lane_sum_splat(v):
    for step in (1, 2, 4, 8):
        v = v + _lane_perm(v, step)
    return v


def _rsqrt_newton(x):
    i = plsc.bitcast(x, jnp.int32)
    i = 0x5F3759DF - (i >> 1)
    y = plsc.bitcast(i, jnp.float32)
    xh = x * 0.5
    for _ in range(2):
        y = y * (1.5 - xh * y * y)
    return y


def _build_prep(rotation_angles, mean_correction, centroids, running_mean):
    c = jnp.cos(rotation_angles)
    s = jnp.sin(rotation_angles)
    cfull = jnp.repeat(c, 2)
    gfull = jnp.stack([-s, s], axis=-1).reshape(-1)
    mean = running_mean + mean_correction
    bounds = (centroids[:-1] + centroids[1:]) * 0.5
    bvec = jnp.concatenate([bounds, jnp.zeros((_L - _NLEV + 1,))])
    lvec = jnp.concatenate([centroids, jnp.zeros((_L - _NLEV,))])
    b3 = jnp.full((_L,), bounds[_NLEV // 2 - 1])
    return jnp.concatenate([cfull, gfull, mean, bvec, lvec, b3]).astype(jnp.float32)


def _make_sc_kernel(n_tokens, block_rows):
    rows_per_w = n_tokens // _NW
    n_blocks = rows_per_w // block_rows

    @functools.partial(
        pl.kernel,
        out_type=jax.ShapeDtypeStruct((n_tokens, _D), jnp.float32),
        mesh=plsc.VectorSubcoreMesh(core_axis_name="c", subcore_axis_name="s"),
        compiler_params=pltpu.CompilerParams(needs_layout_passes=False),
        scratch_types=[
            pltpu.VMEM((block_rows, _D), jnp.float32),
            pltpu.VMEM((block_rows, _D), jnp.float32),
            pltpu.VMEM((_PREP_LEN,), jnp.float32),
        ],
    )
    def sc_kernel(x_hbm, prep_hbm, out_hbm, in_v, out_v, prep_v):
        wid = lax.axis_index("s") * _NC + lax.axis_index("c")
        base = wid * rows_per_w
        pltpu.sync_copy(prep_hbm, prep_v)

        def row_body(r):
            xs = []
            acc = None
            for k in range(_KPR):
                mk = prep_v[pl.ds(_OFF_MEAN + _L * k, _L)]
                v = in_v[r, pl.ds(_L * k, _L)] - mk
                xs.append(v)
                acc = v * v if acc is None else acc + v * v
            totv = jnp.maximum(_lane_sum_splat(acc), 1e-16)
            inv = _rsqrt_newton(totv)
            norm = totv * inv
            bvec = prep_v[pl.ds(_OFF_BVEC, _L)]
            lvec = prep_v[pl.ds(_OFF_LVEC, _L)]
            b3 = prep_v[pl.ds(_OFF_B3, _L)]
            zero = jnp.zeros((_L,), jnp.int32)
            for k in range(_KPR):
                ck = prep_v[pl.ds(_OFF_C + _L * k, _L)]
                gk = prep_v[pl.ds(_OFF_G + _L * k, _L)]
                mk = prep_v[pl.ds(_OFF_MEAN + _L * k, _L)]
                v = xs[k] * inv
                w = _swap_pairs(v)
                rot = ck * v + gk * w
                i0 = jnp.where(rot > b3, jnp.full((_L,), 4, jnp.int32), zero)
                b1v = _lane_gather(bvec, i0 + 1)
                i1 = i0 + jnp.where(rot > b1v, jnp.full((_L,), 2, jnp.int32), zero)
                b2v = _lane_gather(bvec, i1)
                i2 = i1 + jnp.where(rot > b2v, jnp.full((_L,), 1, jnp.int32), zero)
                q = _lane_gather(lvec, i2)
                sq = _swap_pairs(q)
                recon = ck * q - gk * sq
                out_v[r, pl.ds(_L * k, _L)] = recon * norm + mk

        def blk_body(b, carry):
            r0 = base + b * block_rows
            pltpu.sync_copy(x_hbm.at[pl.ds(r0, block_rows)], in_v)
            plsc.parallel_loop(0, block_rows, unroll=4)(row_body)
            pltpu.sync_copy(out_v, out_hbm.at[pl.ds(r0, block_rows)])
            return carry

        lax.fori_loop(0, n_blocks, blk_body, 0)

    return sc_kernel


def kernel(x, rotation_angles, mean_correction, centroids, running_mean):
    x = x.astype(jnp.float32)
    n_tokens = x.shape[0]
    prep = _build_prep(rotation_angles, mean_correction, centroids, running_mean)
    return _make_sc_kernel(n_tokens, 128)(x, prep)


# speedup vs baseline: 7.9159x; 2.6833x over previous
import functools

import jax
import jax.numpy as jnp
from jax import lax
from jax.experimental import pallas as pl
from jax.experimental.pallas import tpu as pltpu
from jax.experimental.pallas import tpu_sc as plsc

_NC = 2
_NS = 16
_NW = _NC * _NS
_L = 16
_D = 128
_KPR = _D // _L
_NLEV = 8

_OFF_C = 0
_OFF_G = 128
_OFF_MEAN = 256
_OFF_BVEC = 384
_OFF_LVEC = 400
_OFF_B3 = 416
_PREP_LEN = 432


def _lane_gather(v, idx):
    return lax.gather(
        v, idx[:, None],
        lax.GatherDimensionNumbers(
            offset_dims=(), collapsed_slice_dims=(0,), start_index_map=(0,)),
        (1,), mode=lax.GatherScatterMode.PROMISE_IN_BOUNDS)


def _lane_perm(v, xor_mask):
    return _lane_gather(v, lax.iota(jnp.int32, _L) ^ xor_mask)


def _swap_pairs(v):
    return _lane_perm(v, 1)


def _lane_sum_splat(v):
    for step in (1, 2, 4, 8):
        v = v + _lane_perm(v, step)
    return v


def _rsqrt_newton(x):
    i = plsc.bitcast(x, jnp.int32)
    i = 0x5F3759DF - (i >> 1)
    y = plsc.bitcast(i, jnp.float32)
    xh = x * 0.5
    for _ in range(2):
        y = y * (1.5 - xh * y * y)
    return y


def _build_prep(rotation_angles, mean_correction, centroids, running_mean):
    c = jnp.cos(rotation_angles)
    s = jnp.sin(rotation_angles)
    cfull = jnp.repeat(c, 2)
    gfull = jnp.stack([-s, s], axis=-1).reshape(-1)
    mean = running_mean + mean_correction
    bounds = (centroids[:-1] + centroids[1:]) * 0.5
    bvec = jnp.concatenate([bounds, jnp.zeros((_L - _NLEV + 1,))])
    lvec = jnp.concatenate([centroids, jnp.zeros((_L - _NLEV,))])
    b3 = jnp.full((_L,), bounds[_NLEV // 2 - 1])
    return jnp.concatenate([cfull, gfull, mean, bvec, lvec, b3]).astype(jnp.float32)


def _make_sc_kernel(n_tokens, block_rows):
    rows_per_w = n_tokens // _NW
    n_blocks = rows_per_w // block_rows

    @functools.partial(
        pl.kernel,
        out_type=jax.ShapeDtypeStruct((n_tokens, _D), jnp.float32),
        mesh=plsc.VectorSubcoreMesh(core_axis_name="c", subcore_axis_name="s"),
        compiler_params=pltpu.CompilerParams(needs_layout_passes=False),
        scratch_types=[
            pltpu.VMEM((block_rows, _D), jnp.float32),
            pltpu.VMEM((block_rows, _D), jnp.float32),
            pltpu.VMEM((_PREP_LEN,), jnp.float32),
        ],
    )
    def sc_kernel(x_hbm, prep_hbm, out_hbm, in_v, out_v, prep_v):
        wid = lax.axis_index("s") * _NC + lax.axis_index("c")
        base = wid * rows_per_w
        pltpu.sync_copy(prep_hbm, prep_v)

        K = range(_KPR)

        def row_body(r):
            mks = [prep_v[pl.ds(_OFF_MEAN + _L * k, _L)] for k in K]
            raw = [in_v[r, pl.ds(_L * k, _L)] for k in K]
            xs = [raw[k] - mks[k] for k in K]
            sq = [x * x for x in xs]
            s4 = [sq[2 * i] + sq[2 * i + 1] for i in range(4)]
            acc = (s4[0] + s4[1]) + (s4[2] + s4[3])
            totv = jnp.maximum(_lane_sum_splat(acc), 1e-16)
            inv = _rsqrt_newton(totv)
            norm = totv * inv
            bvec = prep_v[pl.ds(_OFF_BVEC, _L)]
            lvec = prep_v[pl.ds(_OFF_LVEC, _L)]
            b3 = prep_v[pl.ds(_OFF_B3, _L)]
            zero = jnp.zeros((_L,), jnp.int32)
            four = jnp.full((_L,), 4, jnp.int32)
            two = jnp.full((_L,), 2, jnp.int32)
            one = jnp.full((_L,), 1, jnp.int32)
            cks = [prep_v[pl.ds(_OFF_C + _L * k, _L)] for k in K]
            gks = [prep_v[pl.ds(_OFF_G + _L * k, _L)] for k in K]
            vs = [xs[k] * inv for k in K]
            ws = [_swap_pairs(v) for v in vs]
            ra = [cks[k] * vs[k] for k in K]
            rb = [gks[k] * ws[k] for k in K]
            rot = [ra[k] + rb[k] for k in K]
            i0 = [jnp.where(rot[k] > b3, four, zero) for k in K]
            b1 = [_lane_gather(bvec, i0[k] | 1) for k in K]
            i1 = [i0[k] | jnp.where(rot[k] > b1[k], two, zero) for k in K]
            b2 = [_lane_gather(bvec, i1[k]) for k in K]
            i2 = [i1[k] | jnp.where(rot[k] > b2[k], one, zero) for k in K]
            qs = [_lane_gather(lvec, i2[k]) for k in K]
            sqs = [_swap_pairs(q) for q in qs]
            ua = [cks[k] * qs[k] for k in K]
            ub = [gks[k] * sqs[k] for k in K]
            recon = [ua[k] - ub[k] for k in K]
            outs = [recon[k] * norm + mks[k] for k in K]
            for k in K:
                out_v[r, pl.ds(_L * k, _L)] = outs[k]

        def blk_body(b, carry):
            r0 = base + b * block_rows
            pltpu.sync_copy(x_hbm.at[pl.ds(r0, block_rows)], in_v)
            plsc.parallel_loop(0, block_rows, unroll=2)(row_body)
            pltpu.sync_copy(out_v, out_hbm.at[pl.ds(r0, block_rows)])
            return carry

        lax.fori_loop(0, n_blocks, blk_body, 0)

    return sc_kernel


def kernel(x, rotation_angles, mean_correction, centroids, running_mean):
    x = x.astype(jnp.float32)
    n_tokens = x.shape[0]
    prep = _build_prep(rotation_angles, mean_correction, centroids, running_mean)
    return _make_sc_kernel(n_tokens, 128)(x, prep)


# speedup vs baseline: 9.8683x; 1.2466x over previous
import functools

import jax
import jax.numpy as jnp
from jax import lax
from jax.experimental import pallas as pl
from jax.experimental.pallas import tpu as pltpu
from jax.experimental.pallas import tpu_sc as plsc

_NC = 2
_NS = 16
_NW = _NC * _NS
_L = 16
_D = 128
_KPR = _D // _L
_NLEV = 8

_OFF_C = 0
_OFF_G = 128
_OFF_MEAN = 256
_OFF_BVEC = 384
_OFF_LVEC = 400
_OFF_B3 = 416
_PREP_LEN = 432


def _lane_gather(v, idx):
    return lax.gather(
        v, idx[:, None],
        lax.GatherDimensionNumbers(
            offset_dims=(), collapsed_slice_dims=(0,), start_index_map=(0,)),
        (1,), mode=lax.GatherScatterMode.PROMISE_IN_BOUNDS)


def _lane_perm(v, xor_mask):
    return _lane_gather(v, lax.iota(jnp.int32, _L) ^ xor_mask)


def _swap_pairs(v):
    return _lane_perm(v, 1)


def _lane_sum_splat(v):
    for step in (1, 2, 4, 8):
        v = v + _lane_perm(v, step)
    return v


def _rsqrt_newton(x):
    i = plsc.bitcast(x, jnp.int32)
    i = 0x5F3759DF - (i >> 1)
    y = plsc.bitcast(i, jnp.float32)
    xh = x * 0.5
    for _ in range(2):
        y = y * (1.5 - xh * y * y)
    return y


def _build_prep(rotation_angles, mean_correction, centroids, running_mean):
    c = jnp.cos(rotation_angles)
    s = jnp.sin(rotation_angles)
    cfull = jnp.repeat(c, 2)
    gfull = jnp.stack([-s, s], axis=-1).reshape(-1)
    mean = running_mean + mean_correction
    bounds = (centroids[:-1] + centroids[1:]) * 0.5
    bvec = jnp.concatenate([bounds, jnp.zeros((_L - _NLEV + 1,))])
    lvec = jnp.concatenate([centroids, jnp.zeros((_L - _NLEV,))])
    b3 = jnp.full((_L,), bounds[_NLEV // 2 - 1])
    return jnp.concatenate([cfull, gfull, mean, bvec, lvec, b3]).astype(jnp.float32)


def _make_sc_kernel(n_tokens, block_rows):
    rows_per_w = n_tokens // _NW
    n_blocks = rows_per_w // block_rows

    @functools.partial(
        pl.kernel,
        out_type=jax.ShapeDtypeStruct((n_tokens, _D), jnp.float32),
        mesh=plsc.VectorSubcoreMesh(core_axis_name="c", subcore_axis_name="s"),
        compiler_params=pltpu.CompilerParams(needs_layout_passes=False),
        scratch_types=[
            pltpu.VMEM((block_rows, _D), jnp.float32),
            pltpu.VMEM((block_rows, _D), jnp.float32),
            pltpu.VMEM((block_rows, _D), jnp.float32),
            pltpu.VMEM((block_rows, _D), jnp.float32),
            pltpu.VMEM((_PREP_LEN,), jnp.float32),
            pltpu.SemaphoreType.DMA,
            pltpu.SemaphoreType.DMA,
            pltpu.SemaphoreType.DMA,
            pltpu.SemaphoreType.DMA,
        ],
    )
    def sc_kernel(x_hbm, prep_hbm, out_hbm, in0_v, in1_v, out0_v, out1_v,
                  prep_v, in0_s, in1_s, out0_s, out1_s):
        wid = lax.axis_index("s") * _NC + lax.axis_index("c")
        base = wid * rows_per_w
        pltpu.sync_copy(prep_hbm, prep_v)

        K = range(_KPR)

        def make_row_body(in_v, out_v):
            return lambda r: row_calc(in_v, out_v, r)

        def row_calc(in_v, out_v, r):
            mks = [prep_v[pl.ds(_OFF_MEAN + _L * k, _L)] for k in K]
            raw = [in_v[r, pl.ds(_L * k, _L)] for k in K]
            xs = [raw[k] - mks[k] for k in K]
            sq = [x * x for x in xs]
            s4 = [sq[2 * i] + sq[2 * i + 1] for i in range(4)]
            acc = (s4[0] + s4[1]) + (s4[2] + s4[3])
            totv = jnp.maximum(_lane_sum_splat(acc), 1e-16)
            inv = _rsqrt_newton(totv)
            norm = totv * inv
            bvec = prep_v[pl.ds(_OFF_BVEC, _L)]
            lvec = prep_v[pl.ds(_OFF_LVEC, _L)]
            b3 = prep_v[pl.ds(_OFF_B3, _L)]
            zero = jnp.zeros((_L,), jnp.int32)
            four = jnp.full((_L,), 4, jnp.int32)
            two = jnp.full((_L,), 2, jnp.int32)
            one = jnp.full((_L,), 1, jnp.int32)
            cks = [prep_v[pl.ds(_OFF_C + _L * k, _L)] for k in K]
            gks = [prep_v[pl.ds(_OFF_G + _L * k, _L)] for k in K]
            vs = [xs[k] * inv for k in K]
            ws = [_swap_pairs(v) for v in vs]
            ra = [cks[k] * vs[k] for k in K]
            rb = [gks[k] * ws[k] for k in K]
            rot = [ra[k] + rb[k] for k in K]
            i0 = [jnp.where(rot[k] > b3, four, zero) for k in K]
            b1 = [_lane_gather(bvec, i0[k] | 1) for k in K]
            i1 = [i0[k] | jnp.where(rot[k] > b1[k], two, zero) for k in K]
            b2 = [_lane_gather(bvec, i1[k]) for k in K]
            i2 = [i1[k] | jnp.where(rot[k] > b2[k], one, zero) for k in K]
            qs = [_lane_gather(lvec, i2[k]) for k in K]
            sqs = [_swap_pairs(q) for q in qs]
            ua = [cks[k] * qs[k] for k in K]
            ub = [gks[k] * sqs[k] for k in K]
            recon = [ua[k] - ub[k] for k in K]
            outs = [recon[k] * norm + mks[k] for k in K]
            for k in K:
                out_v[r, pl.ds(_L * k, _L)] = outs[k]

        def compute_block(in_v, out_v):
            plsc.parallel_loop(0, block_rows, unroll=2)(make_row_body(in_v, out_v))

        def in_dma(b, buf, sem):
            pltpu.async_copy(
                x_hbm.at[pl.ds(base + b * block_rows, block_rows)], buf, sem)

        def out_dma(b, buf, sem):
            pltpu.async_copy(
                buf, out_hbm.at[pl.ds(base + b * block_rows, block_rows)], sem)

        def wait_in(buf, sem):
            pltpu.make_async_copy(
                x_hbm.at[pl.ds(base, block_rows)], buf, sem).wait()

        def wait_out(buf, sem):
            pltpu.make_async_copy(
                buf, out_hbm.at[pl.ds(base, block_rows)], sem).wait()

        half = n_blocks // 2
        in_dma(0, in0_v, in0_s)
        in_dma(1, in1_v, in1_s)

        def blk_body(j, carry):
            b0 = 2 * j

            wait_in(in0_v, in0_s)

            @pl.when(j > 0)
            def _():
                wait_out(out0_v, out0_s)

            compute_block(in0_v, out0_v)
            out_dma(b0, out0_v, out0_s)

            @pl.when(j < half - 1)
            def _():
                in_dma(b0 + 2, in0_v, in0_s)

            wait_in(in1_v, in1_s)

            @pl.when(j > 0)
            def _():
                wait_out(out1_v, out1_s)

            compute_block(in1_v, out1_v)
            out_dma(b0 + 1, out1_v, out1_s)

            @pl.when(j < half - 1)
            def _():
                in_dma(b0 + 3, in1_v, in1_s)

            return carry

        lax.fori_loop(0, half, blk_body, 0)
        wait_out(out0_v, out0_s)
        wait_out(out1_v, out1_s)

    return sc_kernel


def kernel(x, rotation_angles, mean_correction, centroids, running_mean):
    x = x.astype(jnp.float32)
    n_tokens = x.shape[0]
    prep = _build_prep(rotation_angles, mean_correction, centroids, running_mean)
    return _make_sc_kernel(n_tokens, 128)(x, prep)
